# baseline (speedup 1.0000x reference)
import functools

import jax
import jax.numpy as jnp
from jax import lax
from jax.experimental import pallas as pl
from jax.experimental.pallas import tpu as pltpu
from jax.experimental.pallas import tpu_sc as plsc

_B = 16384
_S = 50
_D = 32
_NW = 32
_ROWS_W = _B // _NW
_CR = 16
_NC = _ROWS_W // _CR
_CI = _CR * _S
_NBUF = 2


def _sc_gather(idx, table):
    mesh = plsc.VectorSubcoreMesh(core_axis_name="c", subcore_axis_name="s")

    @functools.partial(
        pl.kernel,
        mesh=mesh,
        out_type=jax.ShapeDtypeStruct((_S, _D, _B), jnp.float32),
        scratch_types=[
            [pltpu.VMEM((_CI,), jnp.int32) for _ in range(_NBUF)],
            [pltpu.VMEM((_CI, _D), jnp.float32) for _ in range(_NBUF)],
            [pltpu.VMEM((_S, _D, _CR), jnp.float32) for _ in range(_NBUF)],
            [pltpu.SemaphoreType.DMA for _ in range(_NBUF)],
            [pltpu.SemaphoreType.DMA for _ in range(_NBUF)],
            [pltpu.SemaphoreType.DMA for _ in range(_NBUF)],
        ],
        compiler_params=pltpu.CompilerParams(
            use_tc_tiling_on_sc=False, needs_layout_passes=False),
    )
    def k(idx_hbm, table_hbm, out_hbm, idx_v, rows_v, trows_v,
          isem, gsem, ssem):
        wid = lax.axis_index("s") * 2 + lax.axis_index("c")
        b_base = wid * _ROWS_W
        lane = lax.iota(jnp.int32, 16)

        def idxs(i):
            return pl.ds((b_base + i * _CR) * _S, _CI)

        def gather(i, b):
            del i
            pltpu.async_copy(table_hbm.at[idx_v[b]], rows_v[b], gsem[b])

        def transpose(i, b):
            del i
            src = rows_v[b]
            dst = trows_v[b]

            def per_s(s, carry):
                row_ids = lane * _S + s
                for d in range(_D):
                    col_ids = jnp.full((16,), d, jnp.int32)
                    v = plsc.load_gather(src, [row_ids, col_ids])
                    dst[s, d, :] = v
                return carry

            lax.fori_loop(0, _S, per_s, jnp.int32(0))

        def store(i, b):
            pltpu.async_copy(
                trows_v[b], out_hbm.at[:, :, pl.ds(b_base + i * _CR, _CR)],
                ssem[b])

        def store_wait(i, b):
            pltpu.make_async_copy(
                trows_v[b], out_hbm.at[:, :, pl.ds(b_base + i * _CR, _CR)],
                ssem[b]).wait()

        for b in range(_NBUF):
            pltpu.async_copy(idx_hbm.at[idxs(b)], idx_v[b], isem[b])
        pltpu.make_async_copy(idx_hbm.at[idxs(0)], idx_v[0], isem[0]).wait()
        gather(0, 0)

        def pair(t, carry):
            for half in range(2):
                i = 2 * t + half
                b = half
                o = 1 - half
                nxt = i + 1

                def launch_next():
                    pltpu.make_async_copy(idx_hbm.at[idxs(nxt)], idx_v[o],
                                          isem[o]).wait()
                    gather(nxt, o)

                if half == 0:
                    launch_next()
                else:
                    pl.when(t < _NC // 2 - 1)(launch_next)
                pltpu.make_async_copy(table_hbm.at[idx_v[b]], rows_v[b],
                                      gsem[b]).wait()
                pl.when(t >= 1)(lambda: store_wait(i, b))
                transpose(i, b)
                store(i, b)
                def prefetch():
                    pltpu.async_copy(idx_hbm.at[idxs(i + _NBUF)],
                                     idx_v[b], isem[b])

                pl.when(t < _NC // 2 - 1)(prefetch)
            return carry

        lax.fori_loop(0, _NC // 2, pair, jnp.int32(0))
        for i in (_NC - 2, _NC - 1):
            store_wait(i, i % _NBUF)

    return k(idx, table)


def kernel(time_idx, time_embedding):
    idx_flat = time_idx.reshape(-1).astype(jnp.int32)
    out = _sc_gather(idx_flat, time_embedding)
    return jnp.transpose(out, (2, 0, 1))


# speedup vs baseline: 1.2694x; 1.2694x over previous
import functools

import jax
import jax.numpy as jnp
from jax import lax
from jax.experimental import pallas as pl
from jax.experimental.pallas import tpu as pltpu
from jax.experimental.pallas import tpu_sc as plsc

_B = 16384
_S = 50
_D = 32
_NW = 32
_ROWS_W = _B // _NW
_CR = 16
_NC = _ROWS_W // _CR
_CI = _CR * _S
_CRP = _CR + 1
_NBUF = 2


def _sc_gather(idx, table):
    mesh = plsc.VectorSubcoreMesh(core_axis_name="c", subcore_axis_name="s")

    @functools.partial(
        pl.kernel,
        mesh=mesh,
        out_type=jax.ShapeDtypeStruct((_S, _D, _B), jnp.float32),
        scratch_types=[
            [pltpu.VMEM((_CI,), jnp.int32) for _ in range(_NBUF)],
            [pltpu.VMEM((_CI, _D), jnp.float32) for _ in range(_NBUF)],
            [pltpu.VMEM((_S, _D, _CRP), jnp.float32) for _ in range(_NBUF)],
            [pltpu.SemaphoreType.DMA for _ in range(_NBUF)],
            [pltpu.SemaphoreType.DMA for _ in range(_NBUF)],
            [pltpu.SemaphoreType.DMA for _ in range(_NBUF)],
        ],
        compiler_params=pltpu.CompilerParams(
            use_tc_tiling_on_sc=False, needs_layout_passes=False),
    )
    def k(idx_hbm, table_hbm, out_hbm, idx_v, rows_v, trows_v,
          isem, gsem, ssem):
        wid = lax.axis_index("s") * 2 + lax.axis_index("c")
        b_base = wid * _ROWS_W
        lane = lax.iota(jnp.int32, 16)

        def idxs(i):
            return pl.ds((b_base + i * _CR) * _S, _CI)

        def gather(i, b):
            del i
            pltpu.async_copy(table_hbm.at[idx_v[b]], rows_v[b], gsem[b])

        def transpose(i, b):
            del i
            src = rows_v[b]
            dst = trows_v[b]

            def per_s(s, carry):
                sv = jnp.full((16,), 0, jnp.int32) + s
                for bb in range(_CR):
                    bv = jnp.full((16,), bb, jnp.int32)
                    for h in range(2):
                        v = src[bb * _S + s, pl.ds(h * 16, 16)]
                        plsc.store_scatter(dst, [sv, lane + h * 16, bv], v)
                return carry

            lax.fori_loop(0, _S, per_s, jnp.int32(0))

        def store(i, b):
            pltpu.async_copy(
                trows_v[b].at[:, :, pl.ds(0, _CR)],
                out_hbm.at[:, :, pl.ds(b_base + i * _CR, _CR)], ssem[b])

        def store_wait(i, b):
            pltpu.make_async_copy(
                trows_v[b].at[:, :, pl.ds(0, _CR)],
                out_hbm.at[:, :, pl.ds(b_base + i * _CR, _CR)],
                ssem[b]).wait()

        for b in range(_NBUF):
            pltpu.async_copy(idx_hbm.at[idxs(b)], idx_v[b], isem[b])
        pltpu.make_async_copy(idx_hbm.at[idxs(0)], idx_v[0], isem[0]).wait()
        gather(0, 0)

        def pair(t, carry):
            for half in range(2):
                i = 2 * t + half
                b = half
                o = 1 - half
                nxt = i + 1

                def launch_next():
                    pltpu.make_async_copy(idx_hbm.at[idxs(nxt)], idx_v[o],
                                          isem[o]).wait()
                    gather(nxt, o)

                if half == 0:
                    launch_next()
                else:
                    pl.when(t < _NC // 2 - 1)(launch_next)
                pltpu.make_async_copy(table_hbm.at[idx_v[b]], rows_v[b],
                                      gsem[b]).wait()
                pl.when(t >= 1)(lambda: store_wait(i, b))
                transpose(i, b)
                store(i, b)
                def prefetch():
                    pltpu.async_copy(idx_hbm.at[idxs(i + _NBUF)],
                                     idx_v[b], isem[b])

                pl.when(t < _NC // 2 - 1)(prefetch)
            return carry

        lax.fori_loop(0, _NC // 2, pair, jnp.int32(0))
        for i in (_NC - 2, _NC - 1):
            store_wait(i, i % _NBUF)

    return k(idx, table)


def kernel(time_idx, time_embedding):
    idx_flat = time_idx.reshape(-1).astype(jnp.int32)
    out = _sc_gather(idx_flat, time_embedding)
    return jnp.transpose(out, (2, 0, 1))
